# baseline (speedup 1.0000x reference)
import functools
from itertools import combinations

import jax
import jax.numpy as jnp
from jax import lax
from jax.experimental import pallas as pl
from jax.experimental.pallas import tpu as pltpu
from jax.experimental.pallas import tpu_sc as plsc

B, F, D = 16384, 5, 128
L = 16
NDC = D // L

_TRIPLES = list(combinations(range(F), 3))
_PAIRS = sorted({(i, j) for (i, j, _k) in _TRIPLES})

_info = plsc.get_sparse_core_info()
NC, NS = _info.num_cores, _info.num_subcores
NW = NC * NS
RPW = B // NW
CC = 64
GPC = CC // L
NG = RPW // CC


@functools.partial(
    pl.kernel,
    mesh=plsc.VectorSubcoreMesh(core_axis_name="c", subcore_axis_name="s"),
    compiler_params=pltpu.CompilerParams(needs_layout_passes=False),
    out_type=jax.ShapeDtypeStruct((B,), jnp.float32),
    scratch_types=[
        pltpu.VMEM((F * CC, D), jnp.float32),
        pltpu.VMEM((F * CC, D), jnp.float32),
        pltpu.VMEM((len(_TRIPLES), L), jnp.float32),
        pltpu.VMEM((L * L,), jnp.float32),
        pltpu.VMEM((RPW,), jnp.float32),
        pltpu.SemaphoreType.DMA,
        pltpu.SemaphoreType.DMA,
    ],
)
def _sc_triple_sum(xt_hbm, wb_hbm, out_hbm, buf0, buf1, wb_v,
                   tbuf, out_v, sem0, sem1):
    wid = lax.axis_index("s") * NC + lax.axis_index("c")
    base = wid * RPW

    pltpu.sync_copy(wb_hbm, wb_v)
    wv = [wb_v[t] for t in range(len(_TRIPLES))]
    col_idx = lax.iota(jnp.int32, L) * L

    def start_chunk(off, buf, sem):
        for f in range(F):
            pltpu.async_copy(xt_hbm.at[f, pl.ds(off, CC)],
                             buf.at[pl.ds(f * CC, CC)], sem)

    def wait_chunk(buf, sem):
        for f in range(F):
            pltpu.make_async_copy(xt_hbm.at[0, pl.ds(0, CC)],
                                  buf.at[pl.ds(f * CC, CC)], sem).wait()

    def compute_group(buf, g2):
        for r in range(L):
            acc = None
            for c in range(NDC):
                xs = [buf[f * CC + g2 * L + r, pl.ds(c * L, L)]
                      for f in range(F)]
                pp = {ij: xs[ij[0]] * xs[ij[1]] for ij in _PAIRS}
                for k in range(2, F):
                    inner = None
                    for t, (i, j, kk) in enumerate(_TRIPLES):
                        if kk != k:
                            continue
                        term = wv[t] * pp[(i, j)]
                        inner = term if inner is None else inner + term
                    contrib = xs[k] * inner
                    acc = contrib if acc is None else acc + contrib
            tbuf[pl.ds(r * L, L)] = acc
        total = None
        for c in range(L):
            col = plsc.load_gather(tbuf, [col_idx + c])
            total = col if total is None else total + col
        return total

    slots = ((buf0, sem0), (buf1, sem1))
    NSLOT = len(slots)

    for s_ in range(NSLOT):
        start_chunk(base + s_ * CC, slots[s_][0], slots[s_][1])

    def pair_body(i, _):
        c0 = NSLOT * i
        for s_, (buf, sem) in enumerate(slots):
            cidx = c0 + s_
            wait_chunk(buf, sem)

            def group_body(g2, _, buf=buf, cidx=cidx):
                total = compute_group(buf, g2)
                out_v[pl.ds(cidx * CC + g2 * L, L)] = total
                return 0

            lax.fori_loop(0, GPC, group_body, 0)
            nxt = cidx + NSLOT

            @pl.when(nxt < NG)
            def _prefetch(buf=buf, sem=sem, nxt=nxt):
                start_chunk(base + nxt * CC, buf, sem)
        return 0

    lax.fori_loop(0, NG // NSLOT, pair_body, 0)

    pltpu.sync_copy(out_v, out_hbm.at[pl.ds(base, RPW)])


@jax.jit
def kernel(x, w):
    xt = jnp.transpose(x, (1, 0, 2))
    wb = jnp.broadcast_to(w[:, None], (len(_TRIPLES), L))
    out = _sc_triple_sum(xt, wb)
    return out.reshape(B, 1)


# speedup vs baseline: 2.3408x; 2.3408x over previous
import functools
from itertools import combinations

import jax
import jax.numpy as jnp
from jax import lax
from jax.experimental import pallas as pl
from jax.experimental.pallas import tpu as pltpu
from jax.experimental.pallas import tpu_sc as plsc

B, F, D = 16384, 5, 128
L = 16
NDC = D // L

_TRIPLES = list(combinations(range(F), 3))
_PAIRS = sorted({(i, j) for (i, j, _k) in _TRIPLES})

_info = plsc.get_sparse_core_info()
NC, NS = _info.num_cores, _info.num_subcores
NW = NC * NS
RPW = B // NW
CC = 64
GPC = CC // L
NG = RPW // CC


@functools.partial(
    pl.kernel,
    mesh=plsc.VectorSubcoreMesh(core_axis_name="c", subcore_axis_name="s"),
    compiler_params=pltpu.CompilerParams(needs_layout_passes=False),
    out_type=jax.ShapeDtypeStruct((B,), jnp.float32),
    scratch_types=[
        pltpu.VMEM((F * CC, D), jnp.float32),
        pltpu.VMEM((F * CC, D), jnp.float32),
        pltpu.VMEM((len(_TRIPLES), L), jnp.float32),
        pltpu.VMEM((L * L,), jnp.float32),
        pltpu.VMEM((RPW,), jnp.float32),
        pltpu.SemaphoreType.DMA,
        pltpu.SemaphoreType.DMA,
    ],
)
def _sc_triple_sum(xt_hbm, wb_hbm, out_hbm, buf0, buf1, wb_v, tbuf, out_v,
                   sem0, sem1):
    wid = lax.axis_index("s") * NC + lax.axis_index("c")
    base = wid * RPW

    pltpu.sync_copy(wb_hbm, wb_v)
    wv = [wb_v[t] for t in range(len(_TRIPLES))]
    col_idx = lax.iota(jnp.int32, L) * L

    def start_chunk(off, buf, sem):
        for f in range(F):
            pltpu.async_copy(xt_hbm.at[f, pl.ds(off, CC)],
                             buf.at[pl.ds(f * CC, CC)], sem)

    def wait_chunk(buf, sem):
        for f in range(F):
            pltpu.make_async_copy(xt_hbm.at[0, pl.ds(0, CC)],
                                  buf.at[pl.ds(f * CC, CC)], sem).wait()

    def compute_chunk(buf, cidx):
        def group_body(g2, _, buf=buf, cidx=cidx):
            @plsc.parallel_loop(0, L, unroll=1)
            def row_body(rr, buf=buf, g2=g2):
                r = g2 * L + rr
                acc = None
                for c in range(NDC):
                    xs = [buf[f * CC + r, pl.ds(c * L, L)] for f in range(F)]
                    pp = {ij: xs[ij[0]] * xs[ij[1]] for ij in _PAIRS}
                    for k in range(2, F):
                        inner = None
                        for t, (i, j, kk) in enumerate(_TRIPLES):
                            if kk != k:
                                continue
                            term = wv[t] * pp[(i, j)]
                            inner = term if inner is None else inner + term
                        contrib = xs[k] * inner
                        acc = contrib if acc is None else acc + contrib
                tbuf[pl.ds(rr * L, L)] = acc
            total = None
            for c in range(L):
                col = plsc.load_gather(tbuf, [col_idx + c])
                total = col if total is None else total + col
            out_v[pl.ds(cidx * CC + g2 * L, L)] = total
            return 0

        lax.fori_loop(0, GPC, group_body, 0)

    slots = ((buf0, sem0), (buf1, sem1))
    NSLOT = len(slots)

    for s_ in range(NSLOT):
        start_chunk(base + s_ * CC, slots[s_][0], slots[s_][1])

    def pair_body(i, _):
        c0 = NSLOT * i
        for s_, (buf, sem) in enumerate(slots):
            cidx = c0 + s_
            wait_chunk(buf, sem)
            compute_chunk(buf, cidx)
            nxt = cidx + NSLOT

            @pl.when(nxt < NG)
            def _prefetch(buf=buf, sem=sem, nxt=nxt):
                start_chunk(base + nxt * CC, buf, sem)
        return 0

    lax.fori_loop(0, NG // NSLOT, pair_body, 0)

    pltpu.sync_copy(out_v, out_hbm.at[pl.ds(base, RPW)])


@jax.jit
def kernel(x, w):
    xt = jnp.transpose(x, (1, 0, 2))
    wb = jnp.broadcast_to(w[:, None], (len(_TRIPLES), L))
    out = _sc_triple_sum(xt, wb)
    return out.reshape(B, 1)
